# baseline (speedup 1.0000x reference)
import functools

import numpy as np
import jax
import jax.numpy as jnp
from jax import lax
from jax.experimental import pallas as pl
from jax.experimental.pallas import tpu as pltpu
from jax.experimental.pallas import tpu_sc as plsc

_B_SCALE = 1.3195079565048218
_MAX_PARAMS = 2 ** 19
_HASH_LEVEL = 16
_BASE_RES = 16
_N = 262144

_P1 = int(np.int32(np.uint32(2654435761)))
_P2 = int(np.int32(np.uint32(805459861)))


def _level_tables():
    levels = []
    off = 0
    for i in range(_HASH_LEVEL):
        scale = np.float32(_BASE_RES * np.exp(i * np.log(_B_SCALE)) - 1.0)
        res = int(np.ceil(_BASE_RES * np.exp(i * np.log(_B_SCALE)) - 1.0)) + 1
        p = res ** 3
        p = int(p) if p % 8 == 0 else ((p + 7) // 8) * 8
        p = min(_MAX_PARAMS, p)
        direct = res ** 3 <= p
        if not direct:
            assert p == _MAX_PARAMS
        levels.append({"scale": float(scale), "res": res, "off": off,
                       "size": p, "direct": direct})
        off += p
    return levels, off


_LEVELS, _TOTAL = _level_tables()

_NW = 32
_PW = _N // _NW
_C = 128
_NG = _C // 16
_NCHUNK = _PW // _C
_KIDX = 128
_NSTREAM = (8 * _C) // _KIDX


def _make_encoder():
    mesh = plsc.VectorSubcoreMesh(core_axis_name="c", subcore_axis_name="s",
                                  num_cores=2, num_subcores=16)

    @functools.partial(
        pl.kernel,
        out_type=jax.ShapeDtypeStruct((_N * 32,), jnp.float32),
        mesh=mesh,
        scratch_types=[
            pltpu.VMEM((_C,), jnp.float32),
            pltpu.VMEM((_C,), jnp.float32),
            pltpu.VMEM((_C,), jnp.float32),
            pltpu.VMEM((8 * _C,), jnp.int32),
            pltpu.VMEM((8 * _C,), jnp.float32),
            pltpu.VMEM((8 * _C,), jnp.float32),
            pltpu.VMEM((8 * _C,), jnp.float32),
            pltpu.VMEM((_C * 32,), jnp.float32),
            pltpu.SemaphoreType.DMA,
        ],
        compiler_params=pltpu.CompilerParams(needs_layout_passes=False),
    )
    def enc(xs_hbm, ys_hbm, zs_hbm, t0_hbm, t1_hbm, out_hbm,
            xb, yb, zb, idxb, wgtb, g0b, g1b, outb, sem):
        wid = lax.axis_index("s") * 2 + lax.axis_index("c")
        iota = lax.iota(jnp.int32, 16)
        zero16 = jnp.zeros((16,), jnp.int32)
        one16 = jnp.ones((16,), jnp.int32)

        @pl.loop(0, _NCHUNK)
        def _chunk(ci):
            base = wid * _PW + ci * _C
            pltpu.sync_copy(xs_hbm.at[pl.ds(base, _C)], xb)
            pltpu.sync_copy(ys_hbm.at[pl.ds(base, _C)], yb)
            pltpu.sync_copy(zs_hbm.at[pl.ds(base, _C)], zb)

            for lv, L in enumerate(_LEVELS):
                scale = jnp.float32(L["scale"])
                res = L["res"]
                direct = L["direct"]
                size = L["size"]
                offset = L["off"]

                @pl.loop(0, _NG)
                def _phase1(g):
                    s = pl.ds(g * 16, 16)
                    x = xb[s]
                    y = yb[s]
                    z = zb[s]
                    px = x * scale + 0.5
                    py = y * scale + 0.5
                    pz = z * scale + 0.5
                    gx = px.astype(jnp.int32)
                    gy = py.astype(jnp.int32)
                    gz = pz.astype(jnp.int32)
                    fx = px - gx.astype(jnp.float32)
                    fy = py - gy.astype(jnp.float32)
                    fz = pz - gz.astype(jnp.float32)
                    if direct:
                        ax = (gx, gx + 1)
                        ay = (gy * res, gy * res + res)
                        az = (gz * (res * res), gz * (res * res) + res * res)
                    else:
                        ax = (gx, gx + 1)
                        ay = (gy * _P1, gy * _P1 + _P1)
                        az = (gz * _P2, gz * _P2 + _P2)
                    wx = (1.0 - fx, fx)
                    wy = (1.0 - fy, fy)
                    wz = (1.0 - fz, fz)
                    for cz in range(2):
                        for cy in range(2):
                            wyz = wy[cy] * wz[cz]
                            if direct:
                                hyz = ay[cy] + az[cz]
                            else:
                                hyz = ay[cy] ^ az[cz]
                            for cx in range(2):
                                corner = cx + 2 * cy + 4 * cz
                                if direct:
                                    h = hyz + ax[cx]
                                    h = jnp.where(h >= size, h - size, h)
                                else:
                                    h = (hyz ^ ax[cx]) & (size - 1)
                                cs = pl.ds(corner * _C + g * 16, 16)
                                idxb[cs] = h + offset
                                wgtb[cs] = wx[cx] * wyz

                cps = []
                for j in range(_NSTREAM):
                    js = pl.ds(j * _KIDX, _KIDX)
                    cps.append(pltpu.async_copy(
                        t0_hbm.at[idxb.at[js]], g0b.at[js], sem))
                    cps.append(pltpu.async_copy(
                        t1_hbm.at[idxb.at[js]], g1b.at[js], sem))
                for cp in cps:
                    cp.wait()

                @pl.loop(0, _NG)
                def _phase2(g):
                    f0 = jnp.zeros((16,), jnp.float32)
                    f1 = jnp.zeros((16,), jnp.float32)
                    for corner in range(8):
                        cs = pl.ds(corner * _C + g * 16, 16)
                        w = wgtb[cs]
                        f0 = f0 + w * g0b[cs]
                        f1 = f1 + w * g1b[cs]
                    oidx = (g * 16 + iota) * 32 + (2 * lv)
                    plsc.store_scatter(outb, [oidx], f0)
                    plsc.store_scatter(outb, [oidx + 1], f1)

            pltpu.sync_copy(outb, out_hbm.at[pl.ds(base * 32, _C * 32)])

    return enc


_ENC_CACHE = []


def kernel(positions, table):
    if not _ENC_CACHE:
        _ENC_CACHE.append(_make_encoder())
    xs = positions[:, 0]
    ys = positions[:, 1]
    zs = positions[:, 2]
    t2 = table.reshape(_TOTAL, 2)
    t0 = t2[:, 0]
    t1 = t2[:, 1]
    return _ENC_CACHE[0](xs, ys, zs, t0, t1).reshape(_N, 32)


# speedup vs baseline: 1.0276x; 1.0276x over previous
import functools

import numpy as np
import jax
import jax.numpy as jnp
from jax import lax
from jax.experimental import pallas as pl
from jax.experimental.pallas import tpu as pltpu
from jax.experimental.pallas import tpu_sc as plsc

_B_SCALE = 1.3195079565048218
_MAX_PARAMS = 2 ** 19
_HASH_LEVEL = 16
_BASE_RES = 16
_N = 262144

_P1 = int(np.int32(np.uint32(2654435761)))
_P2 = int(np.int32(np.uint32(805459861)))


def _level_tables():
    levels = []
    off = 0
    for i in range(_HASH_LEVEL):
        scale = np.float32(_BASE_RES * np.exp(i * np.log(_B_SCALE)) - 1.0)
        res = int(np.ceil(_BASE_RES * np.exp(i * np.log(_B_SCALE)) - 1.0)) + 1
        p = res ** 3
        p = int(p) if p % 8 == 0 else ((p + 7) // 8) * 8
        p = min(_MAX_PARAMS, p)
        direct = res ** 3 <= p
        if not direct:
            assert p == _MAX_PARAMS
        levels.append({"scale": float(scale), "res": res, "off": off,
                       "size": p, "direct": direct})
        off += p
    return levels, off


_LEVELS, _TOTAL = _level_tables()

_NW = 32
_PW = _N // _NW
_C = 128
_NG = _C // 16
_NCHUNK = _PW // _C
_KIDX = 128
_NSTREAM = (8 * _C) // _KIDX


def _make_encoder():
    mesh = plsc.VectorSubcoreMesh(core_axis_name="c", subcore_axis_name="s",
                                  num_cores=2, num_subcores=16)

    @functools.partial(
        pl.kernel,
        out_type=jax.ShapeDtypeStruct((_N * 32,), jnp.float32),
        mesh=mesh,
        scratch_types=[
            pltpu.VMEM((_C,), jnp.float32),
            pltpu.VMEM((_C,), jnp.float32),
            pltpu.VMEM((_C,), jnp.float32),
            pltpu.VMEM((8 * _C,), jnp.int32),
            pltpu.VMEM((8 * _C,), jnp.float32),
            pltpu.VMEM((8 * _C, 2), jnp.float32),
            pltpu.VMEM((_C * 32,), jnp.float32),
            pltpu.SemaphoreType.DMA,
        ],
        compiler_params=pltpu.CompilerParams(needs_layout_passes=False,
                                             use_tc_tiling_on_sc=False),
    )
    def enc(xs_hbm, ys_hbm, zs_hbm, table_hbm, out_hbm,
            xb, yb, zb, idxb, wgtb, gathb, outb, sem):
        wid = lax.axis_index("s") * 2 + lax.axis_index("c")
        iota = lax.iota(jnp.int32, 16)
        zero16 = jnp.zeros((16,), jnp.int32)
        one16 = jnp.ones((16,), jnp.int32)

        @pl.loop(0, _NCHUNK)
        def _chunk(ci):
            base = wid * _PW + ci * _C
            pltpu.sync_copy(xs_hbm.at[pl.ds(base, _C)], xb)
            pltpu.sync_copy(ys_hbm.at[pl.ds(base, _C)], yb)
            pltpu.sync_copy(zs_hbm.at[pl.ds(base, _C)], zb)

            for lv, L in enumerate(_LEVELS):
                scale = jnp.float32(L["scale"])
                res = L["res"]
                direct = L["direct"]
                size = L["size"]
                offset = L["off"]

                @pl.loop(0, _NG)
                def _phase1(g):
                    s = pl.ds(g * 16, 16)
                    x = xb[s]
                    y = yb[s]
                    z = zb[s]
                    px = x * scale + 0.5
                    py = y * scale + 0.5
                    pz = z * scale + 0.5
                    gx = px.astype(jnp.int32)
                    gy = py.astype(jnp.int32)
                    gz = pz.astype(jnp.int32)
                    fx = px - gx.astype(jnp.float32)
                    fy = py - gy.astype(jnp.float32)
                    fz = pz - gz.astype(jnp.float32)
                    if direct:
                        ax = (gx, gx + 1)
                        ay = (gy * res, gy * res + res)
                        az = (gz * (res * res), gz * (res * res) + res * res)
                    else:
                        ax = (gx, gx + 1)
                        ay = (gy * _P1, gy * _P1 + _P1)
                        az = (gz * _P2, gz * _P2 + _P2)
                    wx = (1.0 - fx, fx)
                    wy = (1.0 - fy, fy)
                    wz = (1.0 - fz, fz)
                    for cz in range(2):
                        for cy in range(2):
                            wyz = wy[cy] * wz[cz]
                            if direct:
                                hyz = ay[cy] + az[cz]
                            else:
                                hyz = ay[cy] ^ az[cz]
                            for cx in range(2):
                                corner = cx + 2 * cy + 4 * cz
                                if direct:
                                    h = hyz + ax[cx]
                                    h = jnp.where(h >= size, h - size, h)
                                else:
                                    h = (hyz ^ ax[cx]) & (size - 1)
                                cs = pl.ds(corner * _C + g * 16, 16)
                                idxb[cs] = h + offset
                                wgtb[cs] = wx[cx] * wyz

                cps = []
                for j in range(_NSTREAM):
                    js = pl.ds(j * _KIDX, _KIDX)
                    cps.append(pltpu.async_copy(
                        table_hbm.at[idxb.at[js]], gathb.at[js], sem))
                for cp in cps:
                    cp.wait()

                @pl.loop(0, _NG)
                def _phase2(g):
                    f0 = jnp.zeros((16,), jnp.float32)
                    f1 = jnp.zeros((16,), jnp.float32)
                    for corner in range(8):
                        cs = pl.ds(corner * _C + g * 16, 16)
                        rows = corner * _C + g * 16 + iota
                        w = wgtb[cs]
                        f0 = f0 + w * plsc.load_gather(gathb, [rows, zero16])
                        f1 = f1 + w * plsc.load_gather(gathb, [rows, one16])
                    oidx = (g * 16 + iota) * 32 + (2 * lv)
                    plsc.store_scatter(outb, [oidx], f0)
                    plsc.store_scatter(outb, [oidx + 1], f1)

            pltpu.sync_copy(outb, out_hbm.at[pl.ds(base * 32, _C * 32)])

    return enc


_ENC_CACHE = []


def kernel(positions, table):
    if not _ENC_CACHE:
        _ENC_CACHE.append(_make_encoder())
    xs = positions[:, 0]
    ys = positions[:, 1]
    zs = positions[:, 2]
    table2 = table.reshape(_TOTAL, 2)
    return _ENC_CACHE[0](xs, ys, zs, table2).reshape(_N, 32)


# speedup vs baseline: 1.0290x; 1.0014x over previous
import functools

import numpy as np
import jax
import jax.numpy as jnp
from jax import lax
from jax.experimental import pallas as pl
from jax.experimental.pallas import tpu as pltpu
from jax.experimental.pallas import tpu_sc as plsc

_B_SCALE = 1.3195079565048218
_MAX_PARAMS = 2 ** 19
_HASH_LEVEL = 16
_BASE_RES = 16
_N = 262144

_P1 = int(np.int32(np.uint32(2654435761)))
_P2 = int(np.int32(np.uint32(805459861)))


def _level_tables():
    levels = []
    off = 0
    for i in range(_HASH_LEVEL):
        scale = np.float32(_BASE_RES * np.exp(i * np.log(_B_SCALE)) - 1.0)
        res = int(np.ceil(_BASE_RES * np.exp(i * np.log(_B_SCALE)) - 1.0)) + 1
        p = res ** 3
        p = int(p) if p % 8 == 0 else ((p + 7) // 8) * 8
        p = min(_MAX_PARAMS, p)
        direct = res ** 3 <= p
        if not direct:
            assert p == _MAX_PARAMS
        levels.append({"scale": float(scale), "res": res, "off": off,
                       "size": p, "direct": direct})
        off += p
    return levels, off


_LEVELS, _TOTAL = _level_tables()

_NW = 32
_PW = _N // _NW
_C = 128
_NG = _C // 16
_NCHUNK = _PW // _C
_KIDX = 256
_NSTREAM = (8 * _C) // _KIDX


def _make_encoder():
    mesh = plsc.VectorSubcoreMesh(core_axis_name="c", subcore_axis_name="s",
                                  num_cores=2, num_subcores=16)

    @functools.partial(
        pl.kernel,
        out_type=jax.ShapeDtypeStruct((_N * 32,), jnp.float32),
        mesh=mesh,
        scratch_types=[
            pltpu.VMEM((_C,), jnp.float32),
            pltpu.VMEM((_C,), jnp.float32),
            pltpu.VMEM((_C,), jnp.float32),
            pltpu.VMEM((8 * _C,), jnp.int32),
            pltpu.VMEM((8 * _C,), jnp.float32),
            pltpu.VMEM((8 * _C, 2), jnp.float32),
            pltpu.VMEM((_C * 32,), jnp.float32),
            pltpu.SemaphoreType.DMA,
        ],
        compiler_params=pltpu.CompilerParams(needs_layout_passes=False,
                                             use_tc_tiling_on_sc=False),
    )
    def enc(xs_hbm, ys_hbm, zs_hbm, table_hbm, out_hbm,
            xb, yb, zb, idxb, wgtb, gathb, outb, sem):
        wid = lax.axis_index("s") * 2 + lax.axis_index("c")
        iota = lax.iota(jnp.int32, 16)
        zero16 = jnp.zeros((16,), jnp.int32)
        one16 = jnp.ones((16,), jnp.int32)

        @pl.loop(0, _NCHUNK)
        def _chunk(ci):
            base = wid * _PW + ci * _C
            pltpu.sync_copy(xs_hbm.at[pl.ds(base, _C)], xb)
            pltpu.sync_copy(ys_hbm.at[pl.ds(base, _C)], yb)
            pltpu.sync_copy(zs_hbm.at[pl.ds(base, _C)], zb)

            for lv, L in enumerate(_LEVELS):
                scale = jnp.float32(L["scale"])
                res = L["res"]
                direct = L["direct"]
                size = L["size"]
                offset = L["off"]

                @pl.loop(0, _NG)
                def _phase1(g):
                    s = pl.ds(g * 16, 16)
                    x = xb[s]
                    y = yb[s]
                    z = zb[s]
                    px = x * scale + 0.5
                    py = y * scale + 0.5
                    pz = z * scale + 0.5
                    gx = px.astype(jnp.int32)
                    gy = py.astype(jnp.int32)
                    gz = pz.astype(jnp.int32)
                    fx = px - gx.astype(jnp.float32)
                    fy = py - gy.astype(jnp.float32)
                    fz = pz - gz.astype(jnp.float32)
                    if direct:
                        ax = (gx, gx + 1)
                        ay = (gy * res, gy * res + res)
                        az = (gz * (res * res), gz * (res * res) + res * res)
                    else:
                        ax = (gx, gx + 1)
                        ay = (gy * _P1, gy * _P1 + _P1)
                        az = (gz * _P2, gz * _P2 + _P2)
                    wx = (1.0 - fx, fx)
                    wy = (1.0 - fy, fy)
                    wz = (1.0 - fz, fz)
                    for cz in range(2):
                        for cy in range(2):
                            wyz = wy[cy] * wz[cz]
                            if direct:
                                hyz = ay[cy] + az[cz]
                            else:
                                hyz = ay[cy] ^ az[cz]
                            for cx in range(2):
                                corner = cx + 2 * cy + 4 * cz
                                if direct:
                                    h = hyz + ax[cx]
                                    h = jnp.where(h >= size, h - size, h)
                                else:
                                    h = (hyz ^ ax[cx]) & (size - 1)
                                cs = pl.ds(corner * _C + g * 16, 16)
                                idxb[cs] = h + offset
                                wgtb[cs] = wx[cx] * wyz

                cps = []
                for j in range(_NSTREAM):
                    js = pl.ds(j * _KIDX, _KIDX)
                    cps.append(pltpu.async_copy(
                        table_hbm.at[idxb.at[js]], gathb.at[js], sem))
                for cp in cps:
                    cp.wait()

                @pl.loop(0, _NG)
                def _phase2(g):
                    f0 = jnp.zeros((16,), jnp.float32)
                    f1 = jnp.zeros((16,), jnp.float32)
                    for corner in range(8):
                        cs = pl.ds(corner * _C + g * 16, 16)
                        rows = corner * _C + g * 16 + iota
                        w = wgtb[cs]
                        f0 = f0 + w * plsc.load_gather(gathb, [rows, zero16])
                        f1 = f1 + w * plsc.load_gather(gathb, [rows, one16])
                    oidx = (g * 16 + iota) * 32 + (2 * lv)
                    plsc.store_scatter(outb, [oidx], f0)
                    plsc.store_scatter(outb, [oidx + 1], f1)

            pltpu.sync_copy(outb, out_hbm.at[pl.ds(base * 32, _C * 32)])

    return enc


_ENC_CACHE = []


def kernel(positions, table):
    if not _ENC_CACHE:
        _ENC_CACHE.append(_make_encoder())
    xs = positions[:, 0]
    ys = positions[:, 1]
    zs = positions[:, 2]
    table2 = table.reshape(_TOTAL, 2)
    return _ENC_CACHE[0](xs, ys, zs, table2).reshape(_N, 32)


# speedup vs baseline: 1.0701x; 1.0399x over previous
import functools

import numpy as np
import jax
import jax.numpy as jnp
from jax import lax
from jax.experimental import pallas as pl
from jax.experimental.pallas import tpu as pltpu
from jax.experimental.pallas import tpu_sc as plsc

_B_SCALE = 1.3195079565048218
_MAX_PARAMS = 2 ** 19
_HASH_LEVEL = 16
_BASE_RES = 16
_N = 262144

_P1 = int(np.int32(np.uint32(2654435761)))
_P2 = int(np.int32(np.uint32(805459861)))


def _level_tables():
    levels = []
    off = 0
    for i in range(_HASH_LEVEL):
        scale = np.float32(_BASE_RES * np.exp(i * np.log(_B_SCALE)) - 1.0)
        res = int(np.ceil(_BASE_RES * np.exp(i * np.log(_B_SCALE)) - 1.0)) + 1
        p = res ** 3
        p = int(p) if p % 8 == 0 else ((p + 7) // 8) * 8
        p = min(_MAX_PARAMS, p)
        direct = res ** 3 <= p
        if not direct:
            assert p == _MAX_PARAMS
        levels.append({"scale": float(scale), "res": res, "off": off,
                       "size": p, "direct": direct})
        off += p
    return levels, off


_LEVELS, _TOTAL = _level_tables()

_NW = 32
_PW = _N // _NW
_C = 256
_NG = _C // 16
_NCHUNK = _PW // _C
_KIDX = 128
_NSTREAM = (8 * _C) // _KIDX


def _make_encoder():
    mesh = plsc.VectorSubcoreMesh(core_axis_name="c", subcore_axis_name="s",
                                  num_cores=2, num_subcores=16)

    @functools.partial(
        pl.kernel,
        out_type=jax.ShapeDtypeStruct((_N * 32,), jnp.float32),
        mesh=mesh,
        scratch_types=[
            pltpu.VMEM((_C,), jnp.float32),
            pltpu.VMEM((_C,), jnp.float32),
            pltpu.VMEM((_C,), jnp.float32),
            pltpu.VMEM((8 * _C,), jnp.int32),
            pltpu.VMEM((8 * _C,), jnp.float32),
            pltpu.VMEM((8 * _C, 2), jnp.float32),
            pltpu.VMEM((_C * 32,), jnp.float32),
            pltpu.SemaphoreType.DMA,
        ],
        compiler_params=pltpu.CompilerParams(needs_layout_passes=False,
                                             use_tc_tiling_on_sc=False),
    )
    def enc(xs_hbm, ys_hbm, zs_hbm, table_hbm, out_hbm,
            xb, yb, zb, idxb, wgtb, gathb, outb, sem):
        wid = lax.axis_index("s") * 2 + lax.axis_index("c")
        iota = lax.iota(jnp.int32, 16)
        zero16 = jnp.zeros((16,), jnp.int32)
        one16 = jnp.ones((16,), jnp.int32)

        def phase1(L):
            scale = jnp.float32(L["scale"])
            res = L["res"]
            direct = L["direct"]
            size = L["size"]
            offset = L["off"]

            @pl.loop(0, _NG)
            def _p1(g):
                s = pl.ds(g * 16, 16)
                px = xb[s] * scale + 0.5
                py = yb[s] * scale + 0.5
                pz = zb[s] * scale + 0.5
                gx = px.astype(jnp.int32)
                gy = py.astype(jnp.int32)
                gz = pz.astype(jnp.int32)
                fx = px - gx.astype(jnp.float32)
                fy = py - gy.astype(jnp.float32)
                fz = pz - gz.astype(jnp.float32)
                if direct:
                    ax = (gx, gx + 1)
                    ay = (gy * res, gy * res + res)
                    az = (gz * (res * res), gz * (res * res) + res * res)
                else:
                    ax = (gx, gx + 1)
                    ay = (gy * _P1, gy * _P1 + _P1)
                    az = (gz * _P2, gz * _P2 + _P2)
                wx = (1.0 - fx, fx)
                wy = (1.0 - fy, fy)
                wz = (1.0 - fz, fz)
                for cz in range(2):
                    for cy in range(2):
                        wyz = wy[cy] * wz[cz]
                        hyz = (ay[cy] + az[cz]) if direct else (ay[cy] ^ az[cz])
                        for cx in range(2):
                            corner = cx + 2 * cy + 4 * cz
                            if direct:
                                h = hyz + ax[cx]
                                h = jnp.where(h >= size, h - size, h)
                            else:
                                h = (hyz ^ ax[cx]) & (size - 1)
                            cs = pl.ds(corner * _C + g * 16, 16)
                            idxb[cs] = h + offset
                            wgtb[cs] = wx[cx] * wyz

        def fire_and_drain():
            cps = []
            for j in range(_NSTREAM):
                js = pl.ds(j * _KIDX, _KIDX)
                cps.append(pltpu.async_copy(table_hbm.at[idxb.at[js]],
                                            gathb.at[js], sem))
            for cp in cps:
                cp.wait()

        def phase2(lv):
            @pl.loop(0, _NG)
            def _p2(g):
                f0 = jnp.zeros((16,), jnp.float32)
                f1 = jnp.zeros((16,), jnp.float32)
                for corner in range(8):
                    cs = pl.ds(corner * _C + g * 16, 16)
                    rows = corner * _C + g * 16 + iota
                    w = wgtb[cs]
                    f0 = f0 + w * plsc.load_gather(gathb, [rows, zero16])
                    f1 = f1 + w * plsc.load_gather(gathb, [rows, one16])
                oidx = (g * 16 + iota) * 32 + (2 * lv)
                plsc.store_scatter(outb, [oidx], f0)
                plsc.store_scatter(outb, [oidx + 1], f1)

        @pl.loop(0, _NCHUNK)
        def _chunk(ci):
            base = wid * _PW + ci * _C
            pltpu.sync_copy(xs_hbm.at[pl.ds(base, _C)], xb)
            pltpu.sync_copy(ys_hbm.at[pl.ds(base, _C)], yb)
            pltpu.sync_copy(zs_hbm.at[pl.ds(base, _C)], zb)

            for lv, L in enumerate(_LEVELS):
                phase1(L)
                fire_and_drain()
                phase2(lv)

            pltpu.sync_copy(outb, out_hbm.at[pl.ds(base * 32, _C * 32)])

    return enc


_ENC_CACHE = []


def kernel(positions, table):
    if not _ENC_CACHE:
        _ENC_CACHE.append(_make_encoder())
    xs = positions[:, 0]
    ys = positions[:, 1]
    zs = positions[:, 2]
    table2 = table.reshape(_TOTAL, 2)
    return _ENC_CACHE[0](xs, ys, zs, table2).reshape(_N, 32)
